# baseline (speedup 1.0000x reference)
import functools

import jax
import jax.numpy as jnp
from jax import lax
from jax.experimental import pallas as pl
from jax.experimental.pallas import tpu as pltpu
from jax.experimental.pallas import tpu_sc as plsc

_NC = 2
_NS = 16
_NW = _NC * _NS
_LANES = 16
_CH = 8


def _make_sc_mailbox_sum(B, K, D):
    assert B % _CH == 0 and D % _LANES == 0
    nch = B // _CH
    ni = -(-nch // _NW)
    if ni % 2:
        ni += 1
    G = D // _LANES
    mesh = plsc.VectorSubcoreMesh(core_axis_name="c", subcore_axis_name="s")

    @functools.partial(
        pl.kernel,
        out_type=jax.ShapeDtypeStruct((B, D), jnp.float32),
        mesh=mesh,
        scratch_types=[
            pltpu.VMEM((_CH, K, D), jnp.float32),
            pltpu.VMEM((_CH, K, D), jnp.float32),
            pltpu.VMEM((_CH, D), jnp.float32),
            pltpu.SemaphoreType.DMA,
            pltpu.SemaphoreType.DMA,
        ],
    )
    def sc_sum(msg_hbm, out_hbm, buf0, buf1, ob, sem0, sem1):
        wid = lax.axis_index("s") * _NC + lax.axis_index("c")

        def start(i, buf, sem):
            c = wid + i * _NW

            @pl.when(c < nch)
            def _():
                pltpu.async_copy(msg_hbm.at[pl.ds(c * _CH, _CH)], buf, sem)

        def process(i, buf, sem):
            c = wid + i * _NW

            @pl.when(c < nch)
            def _():
                pltpu.make_async_copy(
                    msg_hbm.at[pl.ds(c * _CH, _CH)], buf, sem
                ).wait()
                for n in range(_CH):
                    def row(k, accs):
                        return tuple(
                            accs[g] + buf[n, k, pl.ds(g * _LANES, _LANES)]
                            for g in range(G)
                        )

                    accs = tuple(
                        jnp.zeros((_LANES,), jnp.float32) for _ in range(G)
                    )
                    accs = lax.fori_loop(0, K, row, accs, unroll=4)
                    for g in range(G):
                        ob[n, pl.ds(g * _LANES, _LANES)] = accs[g]
                pltpu.sync_copy(ob, out_hbm.at[pl.ds(c * _CH, _CH)])

        start(0, buf0, sem0)

        def pair(p, carry):
            i0 = 2 * p
            start(i0 + 1, buf1, sem1)
            process(i0, buf0, sem0)
            start(i0 + 2, buf0, sem0)
            process(i0 + 1, buf1, sem1)
            return carry

        lax.fori_loop(0, ni // 2, pair, 0)

    return sc_sum


def _mm_body(h_ref, msum_ref, wh_ref, wm_ref, kk_ref, out_ref):
    hk = h_ref[...] * kk_ref[0]
    out_ref[...] = jnp.dot(
        hk, wh_ref[...], preferred_element_type=jnp.float32
    ) + jnp.dot(msum_ref[...], wm_ref[...], preferred_element_type=jnp.float32)


@jax.jit
def kernel(msg, h, W):
    B, K, d_msg = msg.shape
    d_h = h.shape[-1]
    d_out = W.shape[1]
    Wh = W[:d_h]
    Wm = W[d_h:]

    msum = _make_sc_mailbox_sum(B, K, d_msg)(msg)

    blk = 2000
    kk = jnp.full((1,), jnp.float32(K))
    return pl.pallas_call(
        _mm_body,
        grid=(B // blk,),
        in_specs=[
            pl.BlockSpec((blk, d_h), lambda i: (i, 0)),
            pl.BlockSpec((blk, d_msg), lambda i: (i, 0)),
            pl.BlockSpec((d_h, d_out), lambda i: (0, 0)),
            pl.BlockSpec((d_msg, d_out), lambda i: (0, 0)),
            pl.BlockSpec(memory_space=pltpu.SMEM),
        ],
        out_specs=pl.BlockSpec((blk, d_out), lambda i: (i, 0)),
        out_shape=jax.ShapeDtypeStruct((B, d_out), jnp.float32),
    )(h, msum, Wh, Wm, kk)


# speedup vs baseline: 1.1920x; 1.1920x over previous
import functools

import jax
import jax.numpy as jnp
from jax import lax
from jax.experimental import pallas as pl
from jax.experimental.pallas import tpu as pltpu
from jax.experimental.pallas import tpu_sc as plsc

_NC = 2
_NS = 16
_NW = _NC * _NS
_LANES = 16
_CH = 8
_SPLIT = 5600
_TC_BLK = 1400


def _make_sc_mailbox_sum(B, K, D, s0):
    assert B % _CH == 0 and s0 % _CH == 0 and D % _LANES == 0
    c0 = s0 // _CH
    nch = B // _CH
    ni = -(-(nch - c0) // _NW)
    if ni % 2:
        ni += 1
    G = D // _LANES
    mesh = plsc.VectorSubcoreMesh(core_axis_name="c", subcore_axis_name="s")

    @functools.partial(
        pl.kernel,
        out_type=jax.ShapeDtypeStruct((B - s0, D), jnp.float32),
        mesh=mesh,
        scratch_types=[
            pltpu.VMEM((_CH, K, D), jnp.float32),
            pltpu.VMEM((_CH, K, D), jnp.float32),
            pltpu.VMEM((_CH, D), jnp.float32),
            pltpu.SemaphoreType.DMA,
            pltpu.SemaphoreType.DMA,
        ],
    )
    def sc_sum(msg_hbm, out_hbm, buf0, buf1, ob, sem0, sem1):
        wid = lax.axis_index("s") * _NC + lax.axis_index("c")

        def start(i, buf, sem):
            c = c0 + wid + i * _NW

            @pl.when(c < nch)
            def _():
                pltpu.async_copy(msg_hbm.at[pl.ds(c * _CH, _CH)], buf, sem)

        def process(i, buf, sem):
            c = c0 + wid + i * _NW

            @pl.when(c < nch)
            def _():
                pltpu.make_async_copy(
                    msg_hbm.at[pl.ds(c * _CH, _CH)], buf, sem
                ).wait()
                for n in range(_CH):
                    def row(k, accs):
                        return tuple(
                            accs[g] + buf[n, k, pl.ds(g * _LANES, _LANES)]
                            for g in range(G)
                        )

                    accs = tuple(
                        jnp.zeros((_LANES,), jnp.float32) for _ in range(G)
                    )
                    accs = lax.fori_loop(0, K, row, accs, unroll=4)
                    for g in range(G):
                        ob[n, pl.ds(g * _LANES, _LANES)] = accs[g]
                pltpu.sync_copy(ob, out_hbm.at[pl.ds((c - c0) * _CH, _CH)])

        start(0, buf0, sem0)

        def pair(p, carry):
            i0 = 2 * p
            start(i0 + 1, buf1, sem1)
            process(i0, buf0, sem0)
            start(i0 + 2, buf0, sem0)
            process(i0 + 1, buf1, sem1)
            return carry

        lax.fori_loop(0, ni // 2, pair, 0)

    return sc_sum


def _tc_fused_body(msg_ref, h_ref, wh_ref, wm_ref, out_ref):
    K = msg_ref.shape[1]
    msum = jnp.sum(msg_ref[...], axis=1)
    hk = h_ref[...] * jnp.float32(K)
    out_ref[...] = jnp.dot(
        hk, wh_ref[...], preferred_element_type=jnp.float32
    ) + jnp.dot(msum, wm_ref[...], preferred_element_type=jnp.float32)


def _mm_body(h_ref, msum_ref, wh_ref, wm_ref, kk_ref, out_ref):
    hk = h_ref[...] * kk_ref[0]
    out_ref[...] = jnp.dot(
        hk, wh_ref[...], preferred_element_type=jnp.float32
    ) + jnp.dot(msum_ref[...], wm_ref[...], preferred_element_type=jnp.float32)


@jax.jit
def kernel(msg, h, W):
    B, K, d_msg = msg.shape
    d_h = h.shape[-1]
    d_out = W.shape[1]
    Wh = W[:d_h]
    Wm = W[d_h:]

    msum_sc = _make_sc_mailbox_sum(B, K, d_msg, _SPLIT)(msg)

    out_tc = pl.pallas_call(
        _tc_fused_body,
        grid=(_SPLIT // _TC_BLK,),
        in_specs=[
            pl.BlockSpec((_TC_BLK, K, d_msg), lambda i: (i, 0, 0)),
            pl.BlockSpec((_TC_BLK, d_h), lambda i: (i, 0)),
            pl.BlockSpec((d_h, d_out), lambda i: (0, 0)),
            pl.BlockSpec((d_msg, d_out), lambda i: (0, 0)),
        ],
        out_specs=pl.BlockSpec((_TC_BLK, d_out), lambda i: (i, 0)),
        out_shape=jax.ShapeDtypeStruct((_SPLIT, d_out), jnp.float32),
    )(msg, h, Wh, Wm)

    n_sc = B - _SPLIT
    kk = jnp.full((1,), jnp.float32(K))
    out_sc = pl.pallas_call(
        _mm_body,
        grid=(1,),
        in_specs=[
            pl.BlockSpec((n_sc, d_h), lambda i: (i, 0)),
            pl.BlockSpec((n_sc, d_msg), lambda i: (i, 0)),
            pl.BlockSpec((d_h, d_out), lambda i: (0, 0)),
            pl.BlockSpec((d_msg, d_out), lambda i: (0, 0)),
            pl.BlockSpec(memory_space=pltpu.SMEM),
        ],
        out_specs=pl.BlockSpec((n_sc, d_out), lambda i: (i, 0)),
        out_shape=jax.ShapeDtypeStruct((n_sc, d_out), jnp.float32),
    )(h[_SPLIT:], msum_sc, Wh, Wm, kk)

    return jnp.concatenate([out_tc, out_sc], axis=0)


# speedup vs baseline: 1.2946x; 1.0861x over previous
import functools

import jax
import jax.numpy as jnp
from jax import lax
from jax.experimental import pallas as pl
from jax.experimental.pallas import tpu as pltpu
from jax.experimental.pallas import tpu_sc as plsc

_NC = 2
_NS = 16
_NW = _NC * _NS
_LANES = 16
_CH = 8
_BLK = 1000
_SPLIT = 6000


def _make_sc_mailbox_sum(B, K, D, s0):
    assert B % _CH == 0 and s0 % _CH == 0 and D % _LANES == 0
    c0 = s0 // _CH
    nch = B // _CH
    ni = -(-(nch - c0) // _NW)
    if ni % 2:
        ni += 1
    G = D // _LANES
    mesh = plsc.VectorSubcoreMesh(core_axis_name="c", subcore_axis_name="s")

    @functools.partial(
        pl.kernel,
        out_type=jax.ShapeDtypeStruct((B - s0, D), jnp.float32),
        mesh=mesh,
        scratch_types=[
            pltpu.VMEM((_CH, K, D), jnp.float32),
            pltpu.VMEM((_CH, K, D), jnp.float32),
            pltpu.VMEM((_CH, D), jnp.float32),
            pltpu.SemaphoreType.DMA,
            pltpu.SemaphoreType.DMA,
        ],
    )
    def sc_sum(msg_hbm, out_hbm, buf0, buf1, ob, sem0, sem1):
        wid = lax.axis_index("s") * _NC + lax.axis_index("c")

        def start(i, buf, sem):
            c = c0 + wid + i * _NW

            @pl.when(c < nch)
            def _():
                pltpu.async_copy(msg_hbm.at[pl.ds(c * _CH, _CH)], buf, sem)

        def process(i, buf, sem):
            c = c0 + wid + i * _NW

            @pl.when(c < nch)
            def _():
                pltpu.make_async_copy(
                    msg_hbm.at[pl.ds(c * _CH, _CH)], buf, sem
                ).wait()
                for n in range(_CH):
                    def row(k, accs):
                        return tuple(
                            accs[g] + buf[n, k, pl.ds(g * _LANES, _LANES)]
                            for g in range(G)
                        )

                    accs = tuple(
                        jnp.zeros((_LANES,), jnp.float32) for _ in range(G)
                    )
                    accs = lax.fori_loop(0, K, row, accs, unroll=4)
                    for g in range(G):
                        ob[n, pl.ds(g * _LANES, _LANES)] = accs[g]
                pltpu.sync_copy(ob, out_hbm.at[pl.ds((c - c0) * _CH, _CH)])

        start(0, buf0, sem0)

        def pair(p, carry):
            i0 = 2 * p
            start(i0 + 1, buf1, sem1)
            process(i0, buf0, sem0)
            start(i0 + 2, buf0, sem0)
            process(i0 + 1, buf1, sem1)
            return carry

        lax.fori_loop(0, ni // 2, pair, 0)

    return sc_sum


@jax.jit
def kernel(msg, h, W):
    B, K, d_msg = msg.shape
    d_h = h.shape[-1]
    d_out = W.shape[1]
    Wh = W[:d_h]
    Wm = W[d_h:]
    kf = float(K)

    msum_sc = _make_sc_mailbox_sum(B, K, d_msg, _SPLIT)(msg)

    def tc_fused_body(msg_ref, h_ref, wh_ref, wm_ref, out_ref):
        msum = jnp.sum(msg_ref[...], axis=1)
        out_ref[...] = jnp.dot(
            h_ref[...] * kf, wh_ref[...], preferred_element_type=jnp.float32
        ) + jnp.dot(msum, wm_ref[...], preferred_element_type=jnp.float32)

    out0 = pl.pallas_call(
        tc_fused_body,
        grid=(_SPLIT // _BLK,),
        in_specs=[
            pl.BlockSpec((_BLK, K, d_msg), lambda i: (i, 0, 0)),
            pl.BlockSpec((_BLK, d_h), lambda i: (i, 0)),
            pl.BlockSpec((d_h, d_out), lambda i: (0, 0)),
            pl.BlockSpec((d_msg, d_out), lambda i: (0, 0)),
        ],
        out_specs=pl.BlockSpec((_BLK, d_out), lambda i: (i, 0)),
        out_shape=jax.ShapeDtypeStruct((B, d_out), jnp.float32),
    )(msg, h, Wh, Wm)

    def mm_body(h_ref, msum_ref, wh_ref, wm_ref, prev_ref, out_ref):
        del prev_ref
        out_ref[...] = jnp.dot(
            h_ref[...] * kf, wh_ref[...], preferred_element_type=jnp.float32
        ) + jnp.dot(
            msum_ref[...], wm_ref[...], preferred_element_type=jnp.float32
        )

    off = _SPLIT // _BLK
    out = pl.pallas_call(
        mm_body,
        grid=((B - _SPLIT) // _BLK,),
        in_specs=[
            pl.BlockSpec((_BLK, d_h), lambda i: (i + off, 0)),
            pl.BlockSpec((_BLK, d_msg), lambda i: (i, 0)),
            pl.BlockSpec((d_h, d_out), lambda i: (0, 0)),
            pl.BlockSpec((d_msg, d_out), lambda i: (0, 0)),
            pl.BlockSpec(memory_space=pl.ANY),
        ],
        out_specs=pl.BlockSpec((_BLK, d_out), lambda i: (i + off, 0)),
        out_shape=jax.ShapeDtypeStruct((B, d_out), jnp.float32),
        input_output_aliases={4: 0},
    )(h, msum_sc, Wh, Wm, out0)

    return out


# speedup vs baseline: 1.3717x; 1.0596x over previous
import functools

import jax
import jax.numpy as jnp
from jax import lax
from jax.experimental import pallas as pl
from jax.experimental.pallas import tpu as pltpu
from jax.experimental.pallas import tpu_sc as plsc

_NC = 2
_NS = 16
_NW = _NC * _NS
_LANES = 16
_CH = 8
_BLK = 1000
_SPLIT = 8000


def _make_sc_mailbox_sum(B, K, D, s0):
    assert B % _CH == 0 and s0 % _CH == 0 and D % _LANES == 0
    c0 = s0 // _CH
    nch = B // _CH
    ni = -(-(nch - c0) // _NW)
    if ni % 2:
        ni += 1
    G = D // _LANES
    mesh = plsc.VectorSubcoreMesh(core_axis_name="c", subcore_axis_name="s")

    @functools.partial(
        pl.kernel,
        out_type=jax.ShapeDtypeStruct((B - s0, D), jnp.float32),
        mesh=mesh,
        scratch_types=[
            pltpu.VMEM((_CH, K, D), jnp.float32),
            pltpu.VMEM((_CH, K, D), jnp.float32),
            pltpu.VMEM((_CH, D), jnp.float32),
            pltpu.SemaphoreType.DMA,
            pltpu.SemaphoreType.DMA,
        ],
    )
    def sc_sum(msg_hbm, out_hbm, buf0, buf1, ob, sem0, sem1):
        wid = lax.axis_index("s") * _NC + lax.axis_index("c")

        def start(i, buf, sem):
            c = c0 + wid + i * _NW

            @pl.when(c < nch)
            def _():
                pltpu.async_copy(msg_hbm.at[pl.ds(c * _CH, _CH)], buf, sem)

        def process(i, buf, sem):
            c = c0 + wid + i * _NW

            @pl.when(c < nch)
            def _():
                pltpu.make_async_copy(
                    msg_hbm.at[pl.ds(c * _CH, _CH)], buf, sem
                ).wait()
                for n in range(_CH):
                    def row(k, accs):
                        return tuple(
                            accs[g] + buf[n, k, pl.ds(g * _LANES, _LANES)]
                            for g in range(G)
                        )

                    accs = tuple(
                        jnp.zeros((_LANES,), jnp.float32) for _ in range(G)
                    )
                    accs = lax.fori_loop(0, K, row, accs, unroll=4)
                    for g in range(G):
                        ob[n, pl.ds(g * _LANES, _LANES)] = accs[g]
                pltpu.sync_copy(ob, out_hbm.at[pl.ds((c - c0) * _CH, _CH)])

        start(0, buf0, sem0)

        def pair(p, carry):
            i0 = 2 * p
            start(i0 + 1, buf1, sem1)
            process(i0, buf0, sem0)
            start(i0 + 2, buf0, sem0)
            process(i0 + 1, buf1, sem1)
            return carry

        lax.fori_loop(0, ni // 2, pair, 0)

    return sc_sum


@jax.jit
def kernel(msg, h, W):
    B, K, d_msg = msg.shape
    d_h = h.shape[-1]
    d_out = W.shape[1]
    kf = float(K)

    msum_sc = _make_sc_mailbox_sum(B, K, d_msg, _SPLIT)(msg)

    def tc_fused_body(msg_ref, h_ref, w_ref, out_ref):
        w = w_ref[...]
        msum = jnp.sum(msg_ref[...], axis=1)
        out_ref[...] = jnp.dot(
            h_ref[...] * kf, w[:d_h], preferred_element_type=jnp.float32
        ) + jnp.dot(msum, w[d_h:], preferred_element_type=jnp.float32)

    out0 = pl.pallas_call(
        tc_fused_body,
        grid=(_SPLIT // _BLK,),
        in_specs=[
            pl.BlockSpec((_BLK, K, d_msg), lambda i: (i, 0, 0)),
            pl.BlockSpec((_BLK, d_h), lambda i: (i, 0)),
            pl.BlockSpec((d_h + d_msg, d_out), lambda i: (0, 0)),
        ],
        out_specs=pl.BlockSpec((_BLK, d_out), lambda i: (i, 0)),
        out_shape=jax.ShapeDtypeStruct((B, d_out), jnp.float32),
    )(msg, h, W)

    def mm_body(h_ref, msum_ref, w_ref, prev_ref, out_ref):
        del prev_ref
        w = w_ref[...]
        out_ref[...] = jnp.dot(
            h_ref[...] * kf, w[:d_h], preferred_element_type=jnp.float32
        ) + jnp.dot(
            msum_ref[...], w[d_h:], preferred_element_type=jnp.float32
        )

    n_sc = B - _SPLIT
    off = _SPLIT // n_sc
    assert _SPLIT % n_sc == 0
    out = pl.pallas_call(
        mm_body,
        grid=(1,),
        in_specs=[
            pl.BlockSpec((n_sc, d_h), lambda i: (i + off, 0)),
            pl.BlockSpec((n_sc, d_msg), lambda i: (i, 0)),
            pl.BlockSpec((d_h + d_msg, d_out), lambda i: (0, 0)),
            pl.BlockSpec(memory_space=pl.ANY),
        ],
        out_specs=pl.BlockSpec((n_sc, d_out), lambda i: (i + off, 0)),
        out_shape=jax.ShapeDtypeStruct((B, d_out), jnp.float32),
        input_output_aliases={3: 0},
    )(h, msum_sc, W, out0)

    return out


# speedup vs baseline: 1.3761x; 1.0032x over previous
import functools

import jax
import jax.numpy as jnp
from jax import lax
from jax.experimental import pallas as pl
from jax.experimental.pallas import tpu as pltpu
from jax.experimental.pallas import tpu_sc as plsc

_NC = 2
_NS = 16
_NW = _NC * _NS
_LANES = 16
_CH = 8
_BLK = 1000
_SPLIT = 8000


def _make_sc_mailbox_sum(B, K, D, s0):
    assert B % _CH == 0 and s0 % _CH == 0 and D % _LANES == 0
    c0 = s0 // _CH
    nch = B // _CH
    ni = -(-(nch - c0) // _NW)
    if ni % 2:
        ni += 1
    G = D // _LANES
    mesh = plsc.VectorSubcoreMesh(core_axis_name="c", subcore_axis_name="s")

    @functools.partial(
        pl.kernel,
        out_type=jax.ShapeDtypeStruct((B - s0, D), jnp.float32),
        mesh=mesh,
        scratch_types=[
            pltpu.VMEM((_CH, K, D), jnp.float32),
            pltpu.VMEM((_CH, K, D), jnp.float32),
            pltpu.VMEM((_CH, D), jnp.float32),
            pltpu.SemaphoreType.DMA,
            pltpu.SemaphoreType.DMA,
        ],
    )
    def sc_sum(msg_hbm, out_hbm, buf0, buf1, ob, sem0, sem1):
        wid = lax.axis_index("s") * _NC + lax.axis_index("c")

        def start(i, buf, sem):
            c = c0 + wid + i * _NW

            @pl.when(c < nch)
            def _():
                pltpu.async_copy(msg_hbm.at[pl.ds(c * _CH, _CH)], buf, sem)

        def process(i, buf, sem):
            c = c0 + wid + i * _NW

            @pl.when(c < nch)
            def _():
                pltpu.make_async_copy(
                    msg_hbm.at[pl.ds(c * _CH, _CH)], buf, sem
                ).wait()
                for n in range(_CH):
                    def row(k, accs):
                        return tuple(
                            accs[g] + buf[n, k, pl.ds(g * _LANES, _LANES)]
                            for g in range(G)
                        )

                    accs = tuple(
                        jnp.zeros((_LANES,), jnp.float32) for _ in range(G)
                    )
                    accs = lax.fori_loop(0, K, row, accs, unroll=2)
                    for g in range(G):
                        ob[n, pl.ds(g * _LANES, _LANES)] = accs[g]
                pltpu.sync_copy(ob, out_hbm.at[pl.ds((c - c0) * _CH, _CH)])

        start(0, buf0, sem0)

        def pair(p, carry):
            i0 = 2 * p
            start(i0 + 1, buf1, sem1)
            process(i0, buf0, sem0)
            start(i0 + 2, buf0, sem0)
            process(i0 + 1, buf1, sem1)
            return carry

        lax.fori_loop(0, ni // 2, pair, 0)

    return sc_sum


@jax.jit
def kernel(msg, h, W):
    B, K, d_msg = msg.shape
    d_h = h.shape[-1]
    d_out = W.shape[1]
    kf = float(K)

    msum_sc = _make_sc_mailbox_sum(B, K, d_msg, _SPLIT)(msg)

    def tc_fused_body(msg_ref, h_ref, w_ref, out_ref):
        w = w_ref[...]
        msum = jnp.sum(msg_ref[...], axis=1)
        out_ref[...] = jnp.dot(
            h_ref[...] * kf, w[:d_h], preferred_element_type=jnp.float32
        ) + jnp.dot(msum, w[d_h:], preferred_element_type=jnp.float32)

    out0 = pl.pallas_call(
        tc_fused_body,
        grid=(_SPLIT // _BLK,),
        in_specs=[
            pl.BlockSpec((_BLK, K, d_msg), lambda i: (i, 0, 0)),
            pl.BlockSpec((_BLK, d_h), lambda i: (i, 0)),
            pl.BlockSpec((d_h + d_msg, d_out), lambda i: (0, 0)),
        ],
        out_specs=pl.BlockSpec((_BLK, d_out), lambda i: (i, 0)),
        out_shape=jax.ShapeDtypeStruct((B, d_out), jnp.float32),
    )(msg, h, W)

    def mm_body(h_ref, msum_ref, w_ref, prev_ref, out_ref):
        del prev_ref
        w = w_ref[...]
        out_ref[...] = jnp.dot(
            h_ref[...] * kf, w[:d_h], preferred_element_type=jnp.float32
        ) + jnp.dot(
            msum_ref[...], w[d_h:], preferred_element_type=jnp.float32
        )

    n_sc = B - _SPLIT
    off = _SPLIT // n_sc
    assert _SPLIT % n_sc == 0
    out = pl.pallas_call(
        mm_body,
        grid=(1,),
        in_specs=[
            pl.BlockSpec((n_sc, d_h), lambda i: (i + off, 0)),
            pl.BlockSpec((n_sc, d_msg), lambda i: (i, 0)),
            pl.BlockSpec((d_h + d_msg, d_out), lambda i: (0, 0)),
            pl.BlockSpec(memory_space=pl.ANY),
        ],
        out_specs=pl.BlockSpec((n_sc, d_out), lambda i: (i + off, 0)),
        out_shape=jax.ShapeDtypeStruct((B, d_out), jnp.float32),
        input_output_aliases={3: 0},
    )(h, msum_sc, W, out0)

    return out


# speedup vs baseline: 1.3856x; 1.0069x over previous
import functools

import jax
import jax.numpy as jnp
from jax import lax
from jax.experimental import pallas as pl
from jax.experimental.pallas import tpu as pltpu
from jax.experimental.pallas import tpu_sc as plsc

_NC = 2
_NS = 16
_NW = _NC * _NS
_LANES = 16
_CH = 8
_BLK = 1000
_SPLIT = 9000


def _make_sc_mailbox_sum(B, K, D, s0):
    assert B % _CH == 0 and s0 % _CH == 0 and D % _LANES == 0
    c0 = s0 // _CH
    nch = B // _CH
    ni = -(-(nch - c0) // _NW)
    if ni % 2:
        ni += 1
    G = D // _LANES
    mesh = plsc.VectorSubcoreMesh(core_axis_name="c", subcore_axis_name="s")

    @functools.partial(
        pl.kernel,
        out_type=jax.ShapeDtypeStruct((B - s0, D), jnp.float32),
        mesh=mesh,
        scratch_types=[
            pltpu.VMEM((_CH, K, D), jnp.float32),
            pltpu.VMEM((_CH, K, D), jnp.float32),
            pltpu.VMEM((_CH, D), jnp.float32),
            pltpu.SemaphoreType.DMA,
            pltpu.SemaphoreType.DMA,
        ],
    )
    def sc_sum(msg_hbm, out_hbm, buf0, buf1, ob, sem0, sem1):
        wid = lax.axis_index("s") * _NC + lax.axis_index("c")

        def start(i, buf, sem):
            c = c0 + wid + i * _NW

            @pl.when(c < nch)
            def _():
                pltpu.async_copy(msg_hbm.at[pl.ds(c * _CH, _CH)], buf, sem)

        def process(i, buf, sem):
            c = c0 + wid + i * _NW

            @pl.when(c < nch)
            def _():
                pltpu.make_async_copy(
                    msg_hbm.at[pl.ds(c * _CH, _CH)], buf, sem
                ).wait()
                for n in range(_CH):
                    def row(k, accs):
                        return tuple(
                            accs[g] + buf[n, k, pl.ds(g * _LANES, _LANES)]
                            for g in range(G)
                        )

                    accs = tuple(
                        jnp.zeros((_LANES,), jnp.float32) for _ in range(G)
                    )
                    accs = lax.fori_loop(0, K, row, accs, unroll=2)
                    for g in range(G):
                        ob[n, pl.ds(g * _LANES, _LANES)] = accs[g]
                pltpu.sync_copy(ob, out_hbm.at[pl.ds((c - c0) * _CH, _CH)])

        start(0, buf0, sem0)

        def pair(p, carry):
            i0 = 2 * p
            start(i0 + 1, buf1, sem1)
            process(i0, buf0, sem0)
            start(i0 + 2, buf0, sem0)
            process(i0 + 1, buf1, sem1)
            return carry

        lax.fori_loop(0, ni // 2, pair, 0)

    return sc_sum


@jax.jit
def kernel(msg, h, W):
    B, K, d_msg = msg.shape
    d_h = h.shape[-1]
    d_out = W.shape[1]
    kf = float(K)

    msum_sc = _make_sc_mailbox_sum(B, K, d_msg, _SPLIT)(msg)

    def tc_fused_body(msg_ref, h_ref, w_ref, out_ref):
        w = w_ref[...]
        msum = jnp.sum(msg_ref[...], axis=1)
        out_ref[...] = jnp.dot(
            h_ref[...] * kf, w[:d_h], preferred_element_type=jnp.float32
        ) + jnp.dot(msum, w[d_h:], preferred_element_type=jnp.float32)

    out0 = pl.pallas_call(
        tc_fused_body,
        grid=(_SPLIT // _BLK,),
        in_specs=[
            pl.BlockSpec((_BLK, K, d_msg), lambda i: (i, 0, 0)),
            pl.BlockSpec((_BLK, d_h), lambda i: (i, 0)),
            pl.BlockSpec((d_h + d_msg, d_out), lambda i: (0, 0)),
        ],
        out_specs=pl.BlockSpec((_BLK, d_out), lambda i: (i, 0)),
        out_shape=jax.ShapeDtypeStruct((B, d_out), jnp.float32),
    )(msg, h, W)

    def mm_body(h_ref, msum_ref, w_ref, prev_ref, out_ref):
        del prev_ref
        w = w_ref[...]
        out_ref[...] = jnp.dot(
            h_ref[...] * kf, w[:d_h], preferred_element_type=jnp.float32
        ) + jnp.dot(
            msum_ref[...], w[d_h:], preferred_element_type=jnp.float32
        )

    n_sc = B - _SPLIT
    off = _SPLIT // n_sc
    assert _SPLIT % n_sc == 0
    out = pl.pallas_call(
        mm_body,
        grid=(1,),
        in_specs=[
            pl.BlockSpec((n_sc, d_h), lambda i: (i + off, 0)),
            pl.BlockSpec((n_sc, d_msg), lambda i: (i, 0)),
            pl.BlockSpec((d_h + d_msg, d_out), lambda i: (0, 0)),
            pl.BlockSpec(memory_space=pl.ANY),
        ],
        out_specs=pl.BlockSpec((n_sc, d_out), lambda i: (i + off, 0)),
        out_shape=jax.ShapeDtypeStruct((B, d_out), jnp.float32),
        input_output_aliases={3: 0},
    )(h, msum_sc, W, out0)

    return out
